# baseline (speedup 1.0000x reference)
import functools

import jax
import jax.numpy as jnp
from jax import lax
from jax.experimental import pallas as pl
from jax.experimental.pallas import tpu as pltpu
from jax.experimental.pallas import tpu_sc as plsc

VOCAB = 1000000
MAXLEN = 200
EMBED_DIM = 64
BATCH = 4096

NUM_CORES = 2
NUM_SUBCORES = 16
LANES = 16
NW = NUM_CORES * NUM_SUBCORES
ITEMS_PER_W = BATCH // NW
VECS_PER_ROW = EMBED_DIM // LANES
NBUF = 2
NGROUPS = ITEMS_PER_W // NBUF


def _make_kernel():
    mesh = plsc.VectorSubcoreMesh(core_axis_name="c", subcore_axis_name="s")

    @functools.partial(
        pl.kernel,
        out_type=jax.ShapeDtypeStruct((NW, ITEMS_PER_W, MAXLEN, EMBED_DIM),
                                      jnp.float32),
        mesh=mesh,
        scratch_types=[
            pltpu.VMEM((MAXLEN, EMBED_DIM), jnp.float32),
            pltpu.VMEM((ITEMS_PER_W, MAXLEN), jnp.int32),
            pltpu.VMEM((NBUF, MAXLEN, EMBED_DIM), jnp.float32),
            pltpu.VMEM((NBUF, MAXLEN, EMBED_DIM), jnp.float32),
            pltpu.SemaphoreType.DMA,
            pltpu.SemaphoreType.DMA,
            pltpu.SemaphoreType.DMA,
            pltpu.SemaphoreType.DMA,
        ],
        compiler_params=pltpu.CompilerParams(use_tc_tiling_on_sc=False),
    )
    def tok_pos_embed(x_hbm, tok_hbm, pos_hbm, out_hbm,
                      pos_v, idx_v, gbuf, obuf, g0, g1, o0, o1):
        wid = lax.axis_index("s") * NUM_CORES + lax.axis_index("c")
        gsem = (g0, g1)
        osem = (o0, o1)
        pltpu.sync_copy(pos_hbm, pos_v)
        pltpu.sync_copy(x_hbm.at[wid], idx_v)

        def start_gather(i, b):
            pltpu.async_copy(tok_hbm.at[idx_v.at[i]], gbuf.at[b], gsem[b])

        for b in range(NBUF):
            start_gather(b, b)

        def group_body(g, carry):
            for b in range(NBUF):
                i = g * NBUF + b
                pltpu.make_async_copy(
                    tok_hbm.at[idx_v.at[i]], gbuf.at[b], gsem[b]).wait()

                @pl.when(g >= 1)
                def _wait_prev_out():
                    pltpu.make_async_copy(
                        obuf.at[b], out_hbm.at[wid, 0], osem[b]).wait()

                def add_row(r, cr):
                    for cpart in range(VECS_PER_ROW):
                        sl = pl.ds(cpart * LANES, LANES)
                        obuf[b, r, sl] = gbuf[b, r, sl] + pos_v[r, sl]
                    return cr

                lax.fori_loop(0, MAXLEN, add_row, 0, unroll=4)

                @pl.when(g < NGROUPS - 1)
                def _next_gather():
                    start_gather(i + NBUF, b)

                pltpu.async_copy(obuf.at[b], out_hbm.at[wid, i], osem[b])
            return carry

        lax.fori_loop(0, NGROUPS, group_body, 0)
        for b in range(NBUF):
            pltpu.make_async_copy(
                obuf.at[b], out_hbm.at[wid, 0], osem[b]).wait()

    return tok_pos_embed


_kernel_call = _make_kernel()


def kernel(x, token_table, pos_table):
    x_r = x.astype(jnp.int32).reshape(NW, ITEMS_PER_W, MAXLEN)
    out = _kernel_call(x_r, token_table, pos_table)
    return out.reshape(BATCH, MAXLEN, EMBED_DIM)
